# baseline (speedup 1.0000x reference)
import functools

import jax
import jax.numpy as jnp
from jax import lax
from jax.experimental import pallas as pl
from jax.experimental.pallas import tpu as pltpu
from jax.experimental.pallas import tpu_sc as plsc

_N = 8388608
_N_SPECIES = 100
_NC = 2
_NS = 16
_NW = _NC * _NS
_PER_W = _N // _NW
_CHUNK = 16384
_NCHUNK = _PER_W // _CHUNK
_L = 16
_XB = 3
_ZB = 2
_CW = _CHUNK // 4

_mesh = plsc.VectorSubcoreMesh(core_axis_name="c", subcore_axis_name="s")


@functools.partial(
    pl.kernel,
    out_type=jax.ShapeDtypeStruct((_N,), jnp.float32),
    mesh=_mesh,
    scratch_types=[
        pltpu.VMEM((_N_SPECIES,), jnp.float32),
        pltpu.VMEM((_CHUNK,), jnp.float32),
        pltpu.VMEM((_CHUNK,), jnp.float32),
        pltpu.VMEM((_CHUNK,), jnp.float32),
        pltpu.VMEM((_CW,), jnp.int32),
        pltpu.VMEM((_CW,), jnp.int32),
        pltpu.SemaphoreType.DMA,
        pltpu.SemaphoreType.DMA,
        pltpu.SemaphoreType.DMA,
        pltpu.SemaphoreType.DMA,
        pltpu.SemaphoreType.DMA,
        pltpu.SemaphoreType.DMA,
        pltpu.SemaphoreType.DMA,
        pltpu.SemaphoreType.DMA,
    ],
    compiler_params=pltpu.CompilerParams(needs_layout_passes=False),
)
def _offset_kernel(x_hbm, zp_hbm, off_hbm, out_hbm, tab, xv0, xv1, xv2,
                   zv0, zv1, xi0, xi1, xi2, zi0, zi1, xo0, xo1, xo2):
    wid = lax.axis_index("s") * _NC + lax.axis_index("c")
    base = wid * _PER_W
    wbase = wid * (_PER_W // 4)

    tab_desc = pltpu.async_copy(off_hbm, tab, xo0)

    xvs = (xv0, xv1, xv2)
    zvs = (zv0, zv1)
    x_in_sems = (xi0, xi1, xi2)
    z_in_sems = (zi0, zi1)
    out_sems = (xo0, xo1, xo2)
    x_in_descs = [None] * _XB
    z_in_descs = [None] * _ZB
    out_descs = [None] * _XB

    def start_in(g):
        sx = g % _XB
        sz = g % _ZB
        x_in_descs[sx] = pltpu.async_copy(
            x_hbm.at[pl.ds(base + g * _CHUNK, _CHUNK)], xvs[sx],
            x_in_sems[sx])
        z_in_descs[sz] = pltpu.async_copy(
            zp_hbm.at[pl.ds(wbase + g * _CW, _CW)], zvs[sz], z_in_sems[sz])

    start_in(0)
    tab_desc.wait()
    for g in range(_NCHUNK):
        sx = g % _XB
        sz = g % _ZB
        if g + 1 < _NCHUNK:
            nx = (g + 1) % _XB
            if out_descs[nx] is not None:
                out_descs[nx].wait()
            start_in(g + 1)
        x_in_descs[sx].wait()
        z_in_descs[sz].wait()

        @plsc.parallel_loop(0, _CW, step=_L, unroll=4)
        def body(i, sx=sx, sz=sz):
            w = zvs[sz][pl.ds(i, _L)]
            for k in range(4):
                idx = (w >> (8 * k)) & 0xFF
                vals = plsc.load_gather(tab, [idx])
                plsc.addupdate(xvs[sx].at[pl.ds(i * 4 + k * _L, _L)], vals)

        out_descs[sx] = pltpu.async_copy(
            xvs[sx], out_hbm.at[pl.ds(base + g * _CHUNK, _CHUNK)],
            out_sems[sx])

    for d in out_descs:
        if d is not None:
            d.wait()


def kernel(x, Z, offsets):
    z8 = Z.astype(jnp.int8).reshape(-1, 4, _L).transpose(0, 2, 1)
    zp = jax.lax.bitcast_convert_type(z8, jnp.int32).reshape(_N // 4)
    return _offset_kernel(x, zp, offsets)
